# Initial kernel scaffold; baseline (speedup 1.0000x reference)
import jax
import jax.numpy as jnp
from jax.experimental import pallas as pl


def kernel(x_source, x_target, edge_index_s2t, edge_index_t2s, edge_attr_s2t, edge_attr_t2s, params_s2t, params_t2s, W_final, b_final):
    raise NotImplementedError("write your pallas kernel here")


# speedup vs baseline: 19.9163x; 19.9163x over previous
import functools

import jax
import jax.numpy as jnp
from jax import lax
from jax.experimental import pallas as pl
from jax.experimental.pallas import tpu as pltpu
from jax.experimental.pallas import tpu_sc as plsc

LANES = 16
IDXW = 128
CHUNK = 2048
SUBROWS = CHUNK // IDXW
NSUB = 16
NCORES = 2


def _mesh():
    return plsc.VectorSubcoreMesh(core_axis_name="c", subcore_axis_name="s")


@functools.lru_cache(maxsize=None)
def _make_counts_kernel(npr, npad):
    chunks_per_tile = npad // (NSUB * CHUNK)
    wrows = npr // NSUB
    assert npr % (NSUB * 8) == 0 and npad % (NSUB * CHUNK) == 0

    def body(dst_st, dst_ts, out_t, out_s, zbuf, ones_v, didx, acc, _sem):
        c = lax.axis_index("c")
        s = lax.axis_index("s")

        def fill(i, _):
            zbuf[i, :] = jnp.zeros((LANES,), jnp.float32)
            return 0
        lax.fori_loop(0, wrows, fill, 0)

        def fill1(i, _):
            ones_v[i, :] = jnp.ones((LANES,), jnp.float32)
            return 0
        lax.fori_loop(0, IDXW, fill1, 0)

        pltpu.sync_copy(zbuf, acc.at[pl.ds(s * wrows, wrows)])
        plsc.subcore_barrier()

        def do_dir(dst2d):
            def step(k, _):
                rb = (s * chunks_per_tile + k) * SUBROWS
                pltpu.sync_copy(dst2d.at[pl.ds(rb, SUBROWS)], didx)
                for j in range(SUBROWS):
                    pltpu.sync_copy(ones_v, acc.at[didx.at[j]], add=True)
                return 0
            lax.fori_loop(0, chunks_per_tile, step, 0)

        @pl.when(c == 0)
        def _():
            do_dir(dst_st)

        @pl.when(c == 1)
        def _():
            do_dir(dst_ts)

        plsc.subcore_barrier()

        r0 = s * wrows

        @pl.when(c == 0)
        def _():
            pltpu.sync_copy(acc.at[pl.ds(r0, wrows)], zbuf)
            pltpu.sync_copy(zbuf, out_t.at[pl.ds(r0, wrows)])

        @pl.when(c == 1)
        def _():
            pltpu.sync_copy(acc.at[pl.ds(r0, wrows)], zbuf)
            pltpu.sync_copy(zbuf, out_s.at[pl.ds(r0, wrows)])

    return pl.kernel(
        body,
        out_type=[jax.ShapeDtypeStruct((npr, LANES), jnp.float32)] * 2,
        mesh=_mesh(),
        scratch_types=[
            pltpu.VMEM((wrows, LANES), jnp.float32),
            pltpu.VMEM((IDXW, LANES), jnp.float32),
            pltpu.VMEM((SUBROWS, IDXW), jnp.int32),
            pltpu.VMEM_SHARED((npr, LANES), jnp.float32),
            pltpu.SemaphoreType.DMA,
        ],
        compiler_params=pltpu.CompilerParams(use_tc_tiling_on_sc=False),
    )


@functools.lru_cache(maxsize=None)
def _make_agg_kernel(npr, npad):
    chunks_per_tile = npad // (NSUB * CHUNK)
    wrows = npr // NSUB

    def body(h_st, h_ts, src_st, dst_st, src_ts, dst_ts, cnt_t, cnt_s,
             r_t, r_s, out_t, out_s,
             zbuf, sidx, didx, rows, cntv, rv, outv, acc, tbl, sem):
        c = lax.axis_index("c")
        s = lax.axis_index("s")
        r0 = s * wrows

        def fill(i, _):
            zbuf[i, :] = jnp.zeros((LANES,), jnp.float32)
            return 0
        lax.fori_loop(0, wrows, fill, 0)
        pltpu.sync_copy(zbuf, acc.at[pl.ds(r0, wrows)])

        @pl.when(c == 0)
        def _():
            pltpu.sync_copy(h_st.at[pl.ds(r0, wrows)], tbl.at[pl.ds(r0, wrows)])

        @pl.when(c == 1)
        def _():
            pltpu.sync_copy(h_ts.at[pl.ds(r0, wrows)], tbl.at[pl.ds(r0, wrows)])

        plsc.subcore_barrier()

        def do_dir(src2d, dst2d):
            def step(k, _):
                rb = (s * chunks_per_tile + k) * SUBROWS
                pltpu.sync_copy(src2d.at[pl.ds(rb, SUBROWS)], sidx)
                pltpu.sync_copy(dst2d.at[pl.ds(rb, SUBROWS)], didx)
                descs = [
                    pltpu.async_copy(tbl.at[sidx.at[j]],
                                     rows.at[pl.ds(j * IDXW, IDXW)], sem)
                    for j in range(SUBROWS)
                ]
                for d in descs:
                    d.wait()
                for j in range(SUBROWS):
                    pltpu.sync_copy(rows.at[pl.ds(j * IDXW, IDXW)],
                                    acc.at[didx.at[j]], add=True)
                return 0
            lax.fori_loop(0, chunks_per_tile, step, 0)

        @pl.when(c == 0)
        def _():
            do_dir(src_st, dst_st)

        @pl.when(c == 1)
        def _():
            do_dir(src_ts, dst_ts)

        plsc.subcore_barrier()

        def writeback(cnt, r, out):
            pltpu.sync_copy(acc.at[pl.ds(r0, wrows)], zbuf)
            pltpu.sync_copy(cnt.at[pl.ds(r0, wrows)], cntv)
            pltpu.sync_copy(r.at[pl.ds(r0, wrows)], rv)

            def rowfn(i, _):
                a = zbuf[i, :]
                d = jnp.maximum(cntv[i, :], 1.0)
                outv[i, :] = a / d + rv[i, :]
                return 0
            lax.fori_loop(0, wrows, rowfn, 0)
            pltpu.sync_copy(outv, out.at[pl.ds(r0, wrows)])

        @pl.when(c == 0)
        def _():
            writeback(cnt_t, r_t, out_t)

        @pl.when(c == 1)
        def _():
            writeback(cnt_s, r_s, out_s)

    return pl.kernel(
        body,
        out_type=[jax.ShapeDtypeStruct((npr, LANES), jnp.float32)] * 2,
        mesh=_mesh(),
        scratch_types=[
            pltpu.VMEM((wrows, LANES), jnp.float32),
            pltpu.VMEM((SUBROWS, IDXW), jnp.int32),
            pltpu.VMEM((SUBROWS, IDXW), jnp.int32),
            pltpu.VMEM((CHUNK, LANES), jnp.float32),
            pltpu.VMEM((wrows, LANES), jnp.float32),
            pltpu.VMEM((wrows, LANES), jnp.float32),
            pltpu.VMEM((wrows, LANES), jnp.float32),
            pltpu.VMEM_SHARED((npr, LANES), jnp.float32),
            pltpu.VMEM_SHARED((npr, LANES), jnp.float32),
            pltpu.SemaphoreType.DMA,
        ],
        compiler_params=pltpu.CompilerParams(use_tc_tiling_on_sc=False),
    )


@functools.lru_cache(maxsize=None)
def _make_project_kernel(n_rows, din):
    blk = n_rows // 8
    assert n_rows % 8 == 0 and blk % 8 == 0

    def body(xs_ref, xt_ref, ws_ref, wt_ref, bs_ref, bt_ref,
             h_st_ref, r_s_ref, h_ts_ref, r_t_ref):
        ms = jnp.dot(xs_ref[...], ws_ref[...],
                     preferred_element_type=jnp.float32)
        mt = jnp.dot(xt_ref[...], wt_ref[...],
                     preferred_element_type=jnp.float32)
        h_st_ref[...] = ms[:, :LANES]
        r_s_ref[...] = ms[:, LANES:] + bs_ref[...]
        h_ts_ref[...] = mt[:, :LANES]
        r_t_ref[...] = mt[:, LANES:] + bt_ref[...]

    out16 = jax.ShapeDtypeStruct((n_rows, LANES), jnp.float32)
    spec16 = pl.BlockSpec((blk, LANES), lambda i: (i, 0))
    return pl.pallas_call(
        body,
        grid=(n_rows // blk,),
        in_specs=[
            pl.BlockSpec((blk, din), lambda i: (i, 0)),
            pl.BlockSpec((blk, din), lambda i: (i, 0)),
            pl.BlockSpec((din, 2 * LANES), lambda i: (0, 0)),
            pl.BlockSpec((din, 2 * LANES), lambda i: (0, 0)),
            pl.BlockSpec((1, LANES), lambda i: (0, 0)),
            pl.BlockSpec((1, LANES), lambda i: (0, 0)),
        ],
        out_specs=[spec16, spec16, spec16, spec16],
        out_shape=[out16, out16, out16, out16],
    )


@functools.lru_cache(maxsize=None)
def _make_head_kernel(npr, n_s, n_t):
    def body(xs_ref, xt_ref, wf_ref, bf_ref, out_ref):
        rows = lax.broadcasted_iota(jnp.int32, (npr, 1), 0)
        xs = jnp.where(rows < n_s, xs_ref[...], 0.0)
        xt = jnp.where(rows < n_t, xt_ref[...], 0.0)
        tot = (jnp.sum(xs, axis=0, keepdims=True)
               + jnp.sum(xt, axis=0, keepdims=True))
        mean = tot * (1.0 / float(n_s + n_t))
        val = jnp.sum(mean * wf_ref[...], axis=1, keepdims=True)
        out_ref[...] = val + bf_ref[...]

    return pl.pallas_call(
        body,
        out_shape=jax.ShapeDtypeStruct((1, 1), jnp.float32),
    )


def _pad_edges(edge_index, npad, dummy):
    e = edge_index.shape[1]
    src = jnp.concatenate(
        [edge_index[0], jnp.zeros((npad - e,), jnp.int32)])
    dst = jnp.concatenate(
        [edge_index[1], jnp.full((npad - e,), dummy, jnp.int32)])
    return src.reshape(-1, IDXW), dst.reshape(-1, IDXW)


def kernel(x_source, x_target, edge_index_s2t, edge_index_t2s,
           edge_attr_s2t, edge_attr_t2s, params_s2t, params_t2s,
           W_final, b_final):
    n_s, d_feat = x_source.shape
    n_t = x_target.shape[0]
    assert n_s == n_t, "kernel assumes equal bipartite sides"
    n = n_s
    e = edge_index_s2t.shape[1]
    npad = ((e + NSUB * CHUNK - 1) // (NSUB * CHUNK)) * (NSUB * CHUNK)
    npr = ((n + 1 + NSUB * 8 - 1) // (NSUB * 8)) * (NSUB * 8)

    src_st, dst_st = _pad_edges(edge_index_s2t, npad, n)
    src_ts, dst_ts = _pad_edges(edge_index_t2s, npad, n)

    cnt_t, cnt_s = _make_counts_kernel(npr, npad)(dst_st, dst_ts)

    agg = _make_agg_kernel(npr, npad)
    pad_rows = ((0, npr - n), (0, 0))
    xs = jnp.pad(x_source, pad_rows)
    xt = jnp.pad(x_target, pad_rows)
    for l in range(len(params_s2t)):
        Wl_st, bl_st, Wr_st = params_s2t[l]
        Wl_ts, bl_ts, Wr_ts = params_t2s[l]
        din = xs.shape[1]
        ws = jnp.concatenate([Wl_st, Wr_ts], axis=1)
        wt = jnp.concatenate([Wl_ts, Wr_st], axis=1)
        h_st, r_s, h_ts, r_t = _make_project_kernel(npr, din)(
            xs, xt, ws, wt, bl_ts.reshape(1, LANES), bl_st.reshape(1, LANES))
        out_t, out_s = agg(h_st, h_ts, src_st, dst_st, src_ts, dst_ts,
                           cnt_t, cnt_s, r_t, r_s)
        xs, xt = out_s, out_t

    return _make_head_kernel(npr, n_s, n_t)(
        xs, xt, W_final.reshape(1, LANES), b_final.reshape(1, 1))


# speedup vs baseline: 20.4248x; 1.0255x over previous
import functools

import jax
import jax.numpy as jnp
from jax import lax
from jax.experimental import pallas as pl
from jax.experimental.pallas import tpu as pltpu
from jax.experimental.pallas import tpu_sc as plsc

LANES = 16
IDXW = 128
CHUNK = 1024
SUBROWS = CHUNK // IDXW
NSUB = 16
NCORES = 2


def _mesh():
    return plsc.VectorSubcoreMesh(core_axis_name="c", subcore_axis_name="s")


@functools.lru_cache(maxsize=None)
def _make_agg_kernel(npr, npad, with_counts):
    nchunks = npad // (NSUB * CHUNK)
    wrows = npr // NSUB
    assert npr % (NSUB * 8) == 0 and npad % (NSUB * CHUNK) == 0

    def body(h_st, h_ts, eix_st, eix_ts, cnt_t, cnt_s, r_t, r_s,
             out_t, out_s, cnt_t_o, cnt_s_o,
             sdidx, rows, zv, cntv, rv, ones_v, acc, cntacc, tbl,
             sem_i, sem_g, sem_s):
        c = lax.axis_index("c")
        s = lax.axis_index("s")
        r0 = s * wrows

        def fill(i, _):
            zv[i, :] = jnp.zeros((LANES,), jnp.float32)
            return 0
        lax.fori_loop(0, wrows, fill, 0)
        pltpu.sync_copy(zv, acc.at[pl.ds(r0, wrows)])
        if with_counts:
            pltpu.sync_copy(zv, cntacc.at[pl.ds(r0, wrows)])

            def fill1(i, _):
                ones_v[i, :] = jnp.ones((LANES,), jnp.float32)
                return 0
            lax.fori_loop(0, CHUNK, fill1, 0)

        @pl.when(c == 0)
        def _():
            pltpu.sync_copy(h_st.at[pl.ds(r0, wrows)], tbl.at[pl.ds(r0, wrows)])

        @pl.when(c == 1)
        def _():
            pltpu.sync_copy(h_ts.at[pl.ds(r0, wrows)], tbl.at[pl.ds(r0, wrows)])

        plsc.subcore_barrier()

        def fire_idx(eix, k):
            b = k % 2
            return pltpu.async_copy(eix.at[s * nchunks + k],
                                    sdidx.at[pl.ds(2 * b, 2)], sem_i)

        def do_dir(eix):
            d_idx = fire_idx(eix, 0)
            d_scat = []
            for k in range(nchunks):
                b = k % 2
                sidx = sdidx.at[2 * b]
                didx = sdidx.at[2 * b + 1]
                half = rows.at[pl.ds(b * CHUNK, CHUNK)]
                d_idx.wait()
                d_g = pltpu.async_copy(tbl.at[sidx], half, sem_g)
                for d in d_scat:
                    d.wait()
                if k + 1 < nchunks:
                    d_idx = fire_idx(eix, k + 1)
                d_g.wait()
                d_scat = [pltpu.async_copy(half, acc.at[didx], sem_s,
                                           add=True)]
                if with_counts:
                    d_scat.append(pltpu.async_copy(
                        ones_v, cntacc.at[didx], sem_s, add=True))
            for d in d_scat:
                d.wait()

        @pl.when(c == 0)
        def _():
            do_dir(eix_st)

        @pl.when(c == 1)
        def _():
            do_dir(eix_ts)

        plsc.subcore_barrier()

        def writeback(cnt, r, out, cnt_o):
            pltpu.sync_copy(acc.at[pl.ds(r0, wrows)], zv)
            if with_counts:
                pltpu.sync_copy(cntacc.at[pl.ds(r0, wrows)], cntv)
            else:
                pltpu.sync_copy(cnt.at[pl.ds(r0, wrows)], cntv)
            pltpu.sync_copy(r.at[pl.ds(r0, wrows)], rv)

            def rowfn(i, _):
                a = zv[i, :]
                d = jnp.maximum(cntv[i, :], 1.0)
                zv[i, :] = a / d + rv[i, :]
                return 0
            lax.fori_loop(0, wrows, rowfn, 0)
            pltpu.sync_copy(zv, out.at[pl.ds(r0, wrows)])
            if with_counts:
                pltpu.sync_copy(cntv, cnt_o.at[pl.ds(r0, wrows)])

        @pl.when(c == 0)
        def _():
            writeback(cnt_t, r_t, out_t, cnt_t_o)

        @pl.when(c == 1)
        def _():
            writeback(cnt_s, r_s, out_s, cnt_s_o)

    nodef = jax.ShapeDtypeStruct((npr, LANES), jnp.float32)
    n_out = 4 if with_counts else 2
    scratch = [
        pltpu.VMEM((4, CHUNK), jnp.int32),
        pltpu.VMEM((2 * CHUNK, LANES), jnp.float32),
        pltpu.VMEM((wrows, LANES), jnp.float32),
        pltpu.VMEM((wrows, LANES), jnp.float32),
        pltpu.VMEM((wrows, LANES), jnp.float32),
    ]
    if with_counts:
        scratch += [pltpu.VMEM((CHUNK, LANES), jnp.float32)]
    scratch += [pltpu.VMEM_SHARED((npr, LANES), jnp.float32)]
    if with_counts:
        scratch += [pltpu.VMEM_SHARED((npr, LANES), jnp.float32)]
    scratch += [
        pltpu.VMEM_SHARED((npr, LANES), jnp.float32),
        pltpu.SemaphoreType.DMA,
        pltpu.SemaphoreType.DMA,
        pltpu.SemaphoreType.DMA,
    ]

    if with_counts:
        def body_wc(h_st, h_ts, eix_st, eix_ts, r_t, r_s,
                    out_t, out_s, cnt_t_o, cnt_s_o,
                    sdidx, rows, zv, cntv, rv, ones_v, acc, cntacc,
                    tbl, sem_i, sem_g, sem_s):
            body(h_st, h_ts, eix_st, eix_ts, None, None, r_t, r_s,
                 out_t, out_s, cnt_t_o, cnt_s_o,
                 sdidx, rows, zv, cntv, rv, ones_v, acc, cntacc,
                 tbl, sem_i, sem_g, sem_s)
        fn = body_wc
    else:
        def body_nc(h_st, h_ts, eix_st, eix_ts, cnt_t, cnt_s, r_t, r_s,
                    out_t, out_s,
                    sdidx, rows, zv, cntv, rv, acc,
                    tbl, sem_i, sem_g, sem_s):
            body(h_st, h_ts, eix_st, eix_ts, cnt_t, cnt_s, r_t, r_s,
                 out_t, out_s, None, None,
                 sdidx, rows, zv, cntv, rv, None, acc, None,
                 tbl, sem_i, sem_g, sem_s)
        fn = body_nc

    return pl.kernel(
        fn,
        out_type=[nodef] * n_out,
        mesh=_mesh(),
        scratch_types=scratch,
        compiler_params=pltpu.CompilerParams(use_tc_tiling_on_sc=False),
    )


@functools.lru_cache(maxsize=None)
def _make_project_kernel(n_rows, din):
    blk = n_rows // 8
    assert n_rows % 8 == 0 and blk % 8 == 0

    def body(xs_ref, xt_ref, ws_ref, wt_ref, bs_ref, bt_ref,
             h_st_ref, r_s_ref, h_ts_ref, r_t_ref):
        ms = jnp.dot(xs_ref[...], ws_ref[...],
                     preferred_element_type=jnp.float32)
        mt = jnp.dot(xt_ref[...], wt_ref[...],
                     preferred_element_type=jnp.float32)
        h_st_ref[...] = ms[:, :LANES]
        r_s_ref[...] = ms[:, LANES:] + bs_ref[...]
        h_ts_ref[...] = mt[:, :LANES]
        r_t_ref[...] = mt[:, LANES:] + bt_ref[...]

    out16 = jax.ShapeDtypeStruct((n_rows, LANES), jnp.float32)
    spec16 = pl.BlockSpec((blk, LANES), lambda i: (i, 0))
    return pl.pallas_call(
        body,
        grid=(n_rows // blk,),
        in_specs=[
            pl.BlockSpec((blk, din), lambda i: (i, 0)),
            pl.BlockSpec((blk, din), lambda i: (i, 0)),
            pl.BlockSpec((din, 2 * LANES), lambda i: (0, 0)),
            pl.BlockSpec((din, 2 * LANES), lambda i: (0, 0)),
            pl.BlockSpec((1, LANES), lambda i: (0, 0)),
            pl.BlockSpec((1, LANES), lambda i: (0, 0)),
        ],
        out_specs=[spec16, spec16, spec16, spec16],
        out_shape=[out16, out16, out16, out16],
    )


@functools.lru_cache(maxsize=None)
def _make_head_kernel(npr, n_s, n_t):
    def body(xs_ref, xt_ref, wf_ref, bf_ref, out_ref):
        rows = lax.broadcasted_iota(jnp.int32, (npr, 1), 0)
        xs = jnp.where(rows < n_s, xs_ref[...], 0.0)
        xt = jnp.where(rows < n_t, xt_ref[...], 0.0)
        tot = (jnp.sum(xs, axis=0, keepdims=True)
               + jnp.sum(xt, axis=0, keepdims=True))
        mean = tot * (1.0 / float(n_s + n_t))
        val = jnp.sum(mean * wf_ref[...], axis=1, keepdims=True)
        out_ref[...] = val + bf_ref[...]

    return pl.pallas_call(
        body,
        out_shape=jax.ShapeDtypeStruct((1, 1), jnp.float32),
    )


def _pack_edges(edge_index, npad, dummy):
    e = edge_index.shape[1]
    src = jnp.concatenate(
        [edge_index[0], jnp.zeros((npad - e,), jnp.int32)])
    dst = jnp.concatenate(
        [edge_index[1], jnp.full((npad - e,), dummy, jnp.int32)])
    src3 = src.reshape(-1, 1, CHUNK)
    dst3 = dst.reshape(-1, 1, CHUNK)
    return jnp.concatenate([src3, dst3], axis=1)


def kernel(x_source, x_target, edge_index_s2t, edge_index_t2s,
           edge_attr_s2t, edge_attr_t2s, params_s2t, params_t2s,
           W_final, b_final):
    n_s, d_feat = x_source.shape
    n_t = x_target.shape[0]
    assert n_s == n_t, "kernel assumes equal bipartite sides"
    n = n_s
    e = edge_index_s2t.shape[1]
    npad = ((e + NSUB * CHUNK - 1) // (NSUB * CHUNK)) * (NSUB * CHUNK)
    npr = ((n + 1 + NSUB * 8 - 1) // (NSUB * 8)) * (NSUB * 8)

    eix_st = _pack_edges(edge_index_s2t, npad, n)
    eix_ts = _pack_edges(edge_index_t2s, npad, n)

    pad_rows = ((0, npr - n), (0, 0))
    xs = jnp.pad(x_source, pad_rows)
    xt = jnp.pad(x_target, pad_rows)
    cnt_t = cnt_s = None
    for l in range(len(params_s2t)):
        Wl_st, bl_st, Wr_st = params_s2t[l]
        Wl_ts, bl_ts, Wr_ts = params_t2s[l]
        din = xs.shape[1]
        ws = jnp.concatenate([Wl_st, Wr_ts], axis=1)
        wt = jnp.concatenate([Wl_ts, Wr_st], axis=1)
        h_st, r_s, h_ts, r_t = _make_project_kernel(npr, din)(
            xs, xt, ws, wt, bl_ts.reshape(1, LANES), bl_st.reshape(1, LANES))
        if l == 0:
            out_t, out_s, cnt_t, cnt_s = _make_agg_kernel(npr, npad, True)(
                h_st, h_ts, eix_st, eix_ts, r_t, r_s)
        else:
            out_t, out_s = _make_agg_kernel(npr, npad, False)(
                h_st, h_ts, eix_st, eix_ts, cnt_t, cnt_s, r_t, r_s)
        xs, xt = out_s, out_t

    return _make_head_kernel(npr, n_s, n_t)(
        xs, xt, W_final.reshape(1, LANES), b_final.reshape(1, 1))
